# baseline (speedup 1.0000x reference)
import functools

import jax
import jax.numpy as jnp
from jax import lax
from jax.experimental import pallas as pl
from jax.experimental.pallas import tpu as pltpu
from jax.experimental.pallas import tpu_sc as plsc

_VOCAB = 100000
_DIM = 128
_OUT = 32
_B = 4096
_L = 200

_NC = 2
_NS = 16
_NW = _NC * _NS
_IPT = _B // _NW
_LANES = 16


def _bf16_bits(x):
    u = lax.bitcast_convert_type(x, jnp.int32)
    return (u + jnp.int32(0x7FFF) + ((u >> 16) & 1)) >> 16


def _project_body(t_ref, w_ref, o_ref):
    t = t_ref[...]
    dn = (((1,), (1,)), ((), ()))
    a = lax.dot_general(t, w_ref[0:_OUT // 2, :], dimension_numbers=dn,
                        preferred_element_type=jnp.float32)
    b = lax.dot_general(t, w_ref[_OUT // 2:_OUT, :], dimension_numbers=dn,
                        preferred_element_type=jnp.float32)
    o_ref[...] = (_bf16_bits(a) & jnp.int32(0xFFFF)) | (_bf16_bits(b) << 16)


def _project_table(table, W):
    rows_blk = 5000
    grid = _VOCAB // rows_blk
    return pl.pallas_call(
        _project_body,
        grid=(grid,),
        in_specs=[
            pl.BlockSpec((rows_blk, _DIM), lambda i: (i, 0)),
            pl.BlockSpec((_OUT, _DIM), lambda i: (0, 0)),
        ],
        out_specs=pl.BlockSpec((rows_blk, _OUT // 2), lambda i: (i, 0)),
        out_shape=jax.ShapeDtypeStruct((_VOCAB, _OUT // 2), jnp.int32),
    )(table, W)


def _pool_body(xr_hbm, tp_hbm, b_hbm, out_hbm,
               idx_v, buf0, buf1, acc, bias_v, sem0, sem1):
    wid = lax.axis_index("s") * _NC + lax.axis_index("c")
    pltpu.sync_copy(xr_hbm.at[wid], idx_v)
    pltpu.sync_copy(b_hbm, bias_v)

    zero = jnp.zeros((_LANES,), jnp.float32)

    def zr(r, c):
        acc[r, pl.ds(0, _LANES)] = zero
        acc[r, pl.ds(_LANES, _LANES)] = zero
        return c
    lax.fori_loop(0, _IPT, zr, 0, unroll=8)

    bufs = (buf0, buf1)
    sems = (sem0, sem1)

    pltpu.make_async_copy(tp_hbm.at[idx_v.at[0]], buf0, sem0).start()
    pltpu.make_async_copy(tp_hbm.at[idx_v.at[1]], buf1, sem1).start()

    def outer(i, c):
        l0 = i * 2
        for j in range(2):
            l = l0 + j
            buf = bufs[j]
            sem = sems[j]
            pltpu.make_async_copy(tp_hbm.at[idx_v.at[l]], buf, sem).wait()

            def accum(r, cc):
                vi = buf[r, pl.ds(0, _OUT // 2)]
                lo = plsc.bitcast(vi << 16, jnp.float32)
                hi = plsc.bitcast(vi & jnp.int32(-65536), jnp.float32)
                plsc.addupdate(acc.at[r, pl.ds(0, _LANES)], lo)
                plsc.addupdate(acc.at[r, pl.ds(_LANES, _LANES)], hi)
                return cc
            lax.fori_loop(0, _IPT, accum, 0, unroll=8)

            lnext = l + 2

            @pl.when(lnext < _L)
            def _():
                pltpu.make_async_copy(tp_hbm.at[idx_v.at[lnext]], buf, sem).start()
        return c
    lax.fori_loop(0, _L // 2, outer, 0)

    scale = jnp.float32(1.0 / _L)
    blo = bias_v[pl.ds(0, _LANES)]
    bhi = bias_v[pl.ds(_LANES, _LANES)]

    def fin(r, c):
        v0 = acc[r, pl.ds(0, _LANES)] * scale + blo
        v1 = acc[r, pl.ds(_LANES, _LANES)] * scale + bhi
        acc[r, pl.ds(0, _LANES)] = 1.0 / (1.0 + jnp.exp(-v0))
        acc[r, pl.ds(_LANES, _LANES)] = 1.0 / (1.0 + jnp.exp(-v1))
        return c
    lax.fori_loop(0, _IPT, fin, 0, unroll=4)

    pltpu.sync_copy(acc, out_hbm.at[pl.ds(wid * _IPT, _IPT), :])


@functools.partial(
    pl.kernel,
    mesh=plsc.VectorSubcoreMesh(core_axis_name="c", subcore_axis_name="s"),
    compiler_params=pltpu.CompilerParams(
        use_tc_tiling_on_sc=False, needs_layout_passes=False),
    out_type=jax.ShapeDtypeStruct((_B, _OUT), jnp.float32),
    scratch_types=[
        pltpu.VMEM((_L, _IPT), jnp.int32),
        pltpu.VMEM((_IPT, _OUT // 2), jnp.int32),
        pltpu.VMEM((_IPT, _OUT // 2), jnp.int32),
        pltpu.VMEM((_IPT, _OUT), jnp.float32),
        pltpu.VMEM((_OUT,), jnp.float32),
        pltpu.SemaphoreType.DMA,
        pltpu.SemaphoreType.DMA,
    ],
)
def _pool(xr_hbm, tp_hbm, b_hbm, out_hbm,
          idx_v, buf0, buf1, acc, bias_v, sem0, sem1):
    _pool_body(xr_hbm, tp_hbm, b_hbm, out_hbm,
               idx_v, buf0, buf1, acc, bias_v, sem0, sem1)


def kernel(x, table, W, b):
    x = x.astype(jnp.int32)
    tp = _project_table(table, W)
    xr = x.reshape(_NW, _IPT, _L).transpose(0, 2, 1)
    return _pool(xr, tp, b)


# speedup vs baseline: 1.0747x; 1.0747x over previous
import functools

import jax
import jax.numpy as jnp
from jax import lax
from jax.experimental import pallas as pl
from jax.experimental.pallas import tpu as pltpu
from jax.experimental.pallas import tpu_sc as plsc

_VOCAB = 100000
_DIM = 128
_OUT = 32
_B = 4096
_L = 200

_NC = 2
_NS = 16
_NW = _NC * _NS
_IPT = _B // _NW
_LANES = 16


def _bf16_bits(x):
    u = lax.bitcast_convert_type(x, jnp.int32)
    return (u + jnp.int32(0x7FFF) + ((u >> 16) & 1)) >> 16


def _project_body(t_ref, w_ref, o_ref):
    t = t_ref[...]
    dn = (((1,), (1,)), ((), ()))
    a = lax.dot_general(t, w_ref[0:_OUT // 2, :], dimension_numbers=dn,
                        preferred_element_type=jnp.float32)
    b = lax.dot_general(t, w_ref[_OUT // 2:_OUT, :], dimension_numbers=dn,
                        preferred_element_type=jnp.float32)
    o_ref[...] = (_bf16_bits(a) & jnp.int32(0xFFFF)) | (_bf16_bits(b) << 16)


def _project_table(table, W):
    rows_blk = 5000
    grid = _VOCAB // rows_blk
    return pl.pallas_call(
        _project_body,
        grid=(grid,),
        in_specs=[
            pl.BlockSpec((rows_blk, _DIM), lambda i: (i, 0)),
            pl.BlockSpec((_OUT, _DIM), lambda i: (0, 0)),
        ],
        out_specs=pl.BlockSpec((rows_blk, _OUT // 2), lambda i: (i, 0)),
        out_shape=jax.ShapeDtypeStruct((_VOCAB, _OUT // 2), jnp.int32),
    )(table, W)


_NBUF = 4


def _pool_body(xr_hbm, tp_hbm, b_hbm, out_hbm,
               idx_v, bufs, acc, bias_v, sems):
    wid = lax.axis_index("s") * _NC + lax.axis_index("c")
    pltpu.sync_copy(xr_hbm.at[wid], idx_v)
    pltpu.sync_copy(b_hbm, bias_v)

    zero = jnp.zeros((_LANES,), jnp.float32)

    def zr(r, c):
        acc[r, pl.ds(0, _LANES)] = zero
        acc[r, pl.ds(_LANES, _LANES)] = zero
        return c
    lax.fori_loop(0, _IPT, zr, 0, unroll=8)

    for j in range(_NBUF):
        pltpu.make_async_copy(tp_hbm.at[idx_v.at[j]], bufs[j], sems[j]).start()

    def outer(i, c):
        l0 = i * _NBUF
        for j in range(_NBUF):
            l = l0 + j
            buf = bufs[j]
            sem = sems[j]
            pltpu.make_async_copy(tp_hbm.at[idx_v.at[l]], buf, sem).wait()

            def accum(r, cc):
                vi = buf[r, pl.ds(0, _OUT // 2)]
                lo = plsc.bitcast(vi << 16, jnp.float32)
                hi = plsc.bitcast(vi & jnp.int32(-65536), jnp.float32)
                plsc.addupdate(acc.at[r, pl.ds(0, _LANES)], lo)
                plsc.addupdate(acc.at[r, pl.ds(_LANES, _LANES)], hi)
                return cc
            lax.fori_loop(0, _IPT, accum, 0, unroll=8)

            lnext = l + _NBUF

            @pl.when(lnext < _L)
            def _():
                pltpu.make_async_copy(tp_hbm.at[idx_v.at[lnext]], buf, sem).start()
        return c
    lax.fori_loop(0, _L // _NBUF, outer, 0)

    scale = jnp.float32(1.0 / _L)
    blo = bias_v[pl.ds(0, _LANES)]
    bhi = bias_v[pl.ds(_LANES, _LANES)]

    def fin(r, c):
        v0 = acc[r, pl.ds(0, _LANES)] * scale + blo
        v1 = acc[r, pl.ds(_LANES, _LANES)] * scale + bhi
        acc[r, pl.ds(0, _LANES)] = 1.0 / (1.0 + jnp.exp(-v0))
        acc[r, pl.ds(_LANES, _LANES)] = 1.0 / (1.0 + jnp.exp(-v1))
        return c
    lax.fori_loop(0, _IPT, fin, 0, unroll=4)

    pltpu.sync_copy(acc, out_hbm.at[pl.ds(wid * _IPT, _IPT), :])


@functools.partial(
    pl.kernel,
    mesh=plsc.VectorSubcoreMesh(core_axis_name="c", subcore_axis_name="s"),
    compiler_params=pltpu.CompilerParams(
        use_tc_tiling_on_sc=False, needs_layout_passes=False),
    out_type=jax.ShapeDtypeStruct((_B, _OUT), jnp.float32),
    scratch_types=[
        pltpu.VMEM((_L, _IPT), jnp.int32),
        [pltpu.VMEM((_IPT, _OUT // 2), jnp.int32) for _ in range(_NBUF)],
        pltpu.VMEM((_IPT, _OUT), jnp.float32),
        pltpu.VMEM((_OUT,), jnp.float32),
        [pltpu.SemaphoreType.DMA for _ in range(_NBUF)],
    ],
)
def _pool(xr_hbm, tp_hbm, b_hbm, out_hbm,
          idx_v, bufs, acc, bias_v, sems):
    _pool_body(xr_hbm, tp_hbm, b_hbm, out_hbm,
               idx_v, bufs, acc, bias_v, sems)


def kernel(x, table, W, b):
    x = x.astype(jnp.int32)
    tp = _project_table(table, W)
    xr = x.reshape(_NW, _IPT, _L).transpose(0, 2, 1)
    return _pool(xr, tp, b)


# speedup vs baseline: 1.4894x; 1.3859x over previous
import functools

import jax
import jax.numpy as jnp
from jax import lax
from jax.experimental import pallas as pl
from jax.experimental.pallas import tpu as pltpu
from jax.experimental.pallas import tpu_sc as plsc

_VOCAB = 100000
_DIM = 128
_OUT = 32
_B = 4096
_L = 200

_NC = 2
_NS = 16
_NW = _NC * _NS
_IPT = _B // _NW
_LANES = 16


def _bf16_bits(x):
    u = lax.bitcast_convert_type(x, jnp.int32)
    return (u + jnp.int32(0x7FFF) + ((u >> 16) & 1)) >> 16


def _project_body(t_ref, w_ref, o_ref):
    t = t_ref[...]
    dn = (((1,), (1,)), ((), ()))
    o_ref[...] = lax.dot_general(t, w_ref[...], dimension_numbers=dn,
                                 preferred_element_type=jnp.float32)


def _project_table(table, W):
    rows_blk = 5000
    grid = _VOCAB // rows_blk
    return pl.pallas_call(
        _project_body,
        grid=(grid,),
        in_specs=[
            pl.BlockSpec((rows_blk, _DIM), lambda i: (i, 0)),
            pl.BlockSpec((_OUT, _DIM), lambda i: (0, 0)),
        ],
        out_specs=pl.BlockSpec((rows_blk, _OUT), lambda i: (i, 0)),
        out_shape=jax.ShapeDtypeStruct((_VOCAB, _OUT), jnp.float32),
    )(table, W)


_NBUF = 4


def _pool_body(xr_hbm, tp_hbm, b_hbm, out_hbm,
               idx_v, bufs, acc, bias_v, sems):
    wid = lax.axis_index("s") * _NC + lax.axis_index("c")
    pltpu.sync_copy(xr_hbm.at[wid], idx_v)
    pltpu.sync_copy(b_hbm, bias_v)

    zero = jnp.zeros((_LANES,), jnp.float32)

    def zr(r, c):
        acc[r, pl.ds(0, _LANES)] = zero
        acc[r, pl.ds(_LANES, _LANES)] = zero
        return c
    lax.fori_loop(0, _IPT, zr, 0, unroll=8)

    for j in range(_NBUF):
        pltpu.make_async_copy(tp_hbm.at[idx_v.at[j]], bufs[j], sems[j]).start()

    def outer(i, c):
        l0 = i * _NBUF
        for j in range(_NBUF):
            l = l0 + j
            buf = bufs[j]
            sem = sems[j]
            pltpu.make_async_copy(tp_hbm.at[idx_v.at[l]], buf, sem).wait()

            def accum(r, cc):
                plsc.addupdate(acc.at[r, pl.ds(0, _LANES)],
                               buf[r, pl.ds(0, _LANES)])
                plsc.addupdate(acc.at[r, pl.ds(_LANES, _LANES)],
                               buf[r, pl.ds(_LANES, _LANES)])
                return cc
            lax.fori_loop(0, _IPT, accum, 0, unroll=8)

            lnext = l + _NBUF

            @pl.when(lnext < _L)
            def _():
                pltpu.make_async_copy(tp_hbm.at[idx_v.at[lnext]], buf, sem).start()
        return c
    lax.fori_loop(0, _L // _NBUF, outer, 0)

    scale = jnp.float32(1.0 / _L)
    blo = bias_v[pl.ds(0, _LANES)]
    bhi = bias_v[pl.ds(_LANES, _LANES)]

    def fin(r, c):
        v0 = acc[r, pl.ds(0, _LANES)] * scale + blo
        v1 = acc[r, pl.ds(_LANES, _LANES)] * scale + bhi
        acc[r, pl.ds(0, _LANES)] = 1.0 / (1.0 + jnp.exp(-v0))
        acc[r, pl.ds(_LANES, _LANES)] = 1.0 / (1.0 + jnp.exp(-v1))
        return c
    lax.fori_loop(0, _IPT, fin, 0, unroll=4)

    pltpu.sync_copy(acc, out_hbm.at[pl.ds(wid * _IPT, _IPT), :])


@functools.partial(
    pl.kernel,
    mesh=plsc.VectorSubcoreMesh(core_axis_name="c", subcore_axis_name="s"),
    compiler_params=pltpu.CompilerParams(
        use_tc_tiling_on_sc=False, needs_layout_passes=False),
    out_type=jax.ShapeDtypeStruct((_B, _OUT), jnp.float32),
    scratch_types=[
        pltpu.VMEM((_L, _IPT), jnp.int32),
        [pltpu.VMEM((_IPT, _OUT), jnp.float32) for _ in range(_NBUF)],
        pltpu.VMEM((_IPT, _OUT), jnp.float32),
        pltpu.VMEM((_OUT,), jnp.float32),
        [pltpu.SemaphoreType.DMA for _ in range(_NBUF)],
    ],
)
def _pool(xr_hbm, tp_hbm, b_hbm, out_hbm,
          idx_v, bufs, acc, bias_v, sems):
    _pool_body(xr_hbm, tp_hbm, b_hbm, out_hbm,
               idx_v, bufs, acc, bias_v, sems)


def kernel(x, table, W, b):
    x = x.astype(jnp.int32)
    tp = _project_table(table, W)
    xr = x.reshape(_NW, _IPT, _L).transpose(0, 2, 1)
    return _pool(xr, tp, b)


# speedup vs baseline: 2.5026x; 1.6803x over previous
import functools

import jax
import jax.numpy as jnp
from jax import lax
from jax.experimental import pallas as pl
from jax.experimental.pallas import tpu as pltpu
from jax.experimental.pallas import tpu_sc as plsc

_VOCAB = 100000
_DIM = 128
_OUT = 32
_B = 4096
_L = 200

_NC = 2
_NS = 16
_NW = _NC * _NS
_IPT = _B // _NW
_LANES = 16


def _bf16_bits(x):
    u = lax.bitcast_convert_type(x, jnp.int32)
    return (u + jnp.int32(0x7FFF) + ((u >> 16) & 1)) >> 16


_PACK = _DIM // _OUT
_VS = _VOCAB // _PACK


def _project_body(t0, t1, t2, t3, w_ref, o_ref):
    dn = (((1,), (1,)), ((), ()))
    for j, tr in enumerate((t0, t1, t2, t3)):
        o_ref[:, _OUT * j:_OUT * (j + 1)] = lax.dot_general(
            tr[...], w_ref[...], dimension_numbers=dn,
            preferred_element_type=jnp.float32)


def _project_table(table, W):
    rows_blk = 5000
    grid = _VS // rows_blk
    in_specs = [
        pl.BlockSpec((rows_blk, _DIM),
                     (lambda g, jj=j: (g + jj * grid, 0)))
        for j in range(_PACK)
    ]
    in_specs.append(pl.BlockSpec((_OUT, _DIM), lambda g: (0, 0)))
    return pl.pallas_call(
        _project_body,
        grid=(grid,),
        in_specs=in_specs,
        out_specs=pl.BlockSpec((rows_blk, _DIM), lambda g: (g, 0)),
        out_shape=jax.ShapeDtypeStruct((_VS, _DIM), jnp.float32),
    )(table, table, table, table, W)


_NBUF = 4
_LPG = 5


def _pool_body(xr_hbm, tp_hbm, b_hbm, out_hbm,
               idx_v, bufs, acc, bias_v, sems):
    wid = lax.axis_index("s") * _NC + lax.axis_index("c")

    nsteps = _L // _LPG
    prime_span = _NBUF * _LPG * _IPT

    pltpu.sync_copy(xr_hbm.at[wid, pl.ds(0, prime_span)],
                    idx_v.at[pl.ds(0, prime_span)])
    for j in range(_NBUF):
        pltpu.make_async_copy(
            tp_hbm.at[idx_v.at[pl.ds(j * _LPG * _IPT, _LPG * _IPT)]],
            bufs[j], sems[j]).start()
    pltpu.sync_copy(xr_hbm.at[wid, pl.ds(prime_span, _L * _IPT - prime_span)],
                    idx_v.at[pl.ds(prime_span, _L * _IPT - prime_span)])
    pltpu.sync_copy(b_hbm, bias_v)

    zero = jnp.zeros((_LANES,), jnp.float32)

    def zr(r, c):
        acc[r, pl.ds(0, _LANES)] = zero
        acc[r, pl.ds(_LANES, _LANES)] = zero
        return c
    lax.fori_loop(0, _IPT, zr, 0, unroll=8)

    def outer(i, c):
        s0 = i * _NBUF
        for j in range(_NBUF):
            s = s0 + j
            buf = bufs[j]
            sem = sems[j]
            pltpu.make_async_copy(
                tp_hbm.at[idx_v.at[pl.ds(s * _LPG * _IPT, _LPG * _IPT)]],
                buf, sem).wait()

            def accum(r, cc):
                v0 = buf[r, pl.ds(0, _LANES)]
                v1 = buf[r, pl.ds(_LANES, _LANES)]
                for part in range(1, _LPG):
                    rr = part * _IPT + r
                    v0 = v0 + buf[rr, pl.ds(0, _LANES)]
                    v1 = v1 + buf[rr, pl.ds(_LANES, _LANES)]
                plsc.addupdate(acc.at[r, pl.ds(0, _LANES)], v0)
                plsc.addupdate(acc.at[r, pl.ds(_LANES, _LANES)], v1)
                return cc
            lax.fori_loop(0, _IPT, accum, 0, unroll=8)

            snext = s + _NBUF

            @pl.when(snext < nsteps)
            def _():
                pltpu.make_async_copy(
                    tp_hbm.at[idx_v.at[pl.ds(snext * _LPG * _IPT,
                                             _LPG * _IPT)]], buf, sem
                ).start()
        return c
    lax.fori_loop(0, nsteps // _NBUF, outer, 0)

    scale = jnp.float32(1.0 / _L)
    blo = bias_v[pl.ds(0, _LANES)]
    bhi = bias_v[pl.ds(_LANES, _LANES)]

    def fin(r, c):
        v0 = acc[r, pl.ds(0, _LANES)] * scale + blo
        v1 = acc[r, pl.ds(_LANES, _LANES)] * scale + bhi
        acc[r, pl.ds(0, _LANES)] = 1.0 / (1.0 + jnp.exp(-v0))
        acc[r, pl.ds(_LANES, _LANES)] = 1.0 / (1.0 + jnp.exp(-v1))
        return c
    lax.fori_loop(0, _IPT, fin, 0, unroll=4)

    pltpu.sync_copy(acc, out_hbm.at[pl.ds(wid * _IPT, _IPT), :])


@functools.partial(
    pl.kernel,
    mesh=plsc.VectorSubcoreMesh(core_axis_name="c", subcore_axis_name="s"),
    compiler_params=pltpu.CompilerParams(
        use_tc_tiling_on_sc=False, needs_layout_passes=False),
    out_type=jax.ShapeDtypeStruct((_B, _OUT), jnp.float32),
    scratch_types=[
        pltpu.VMEM((_L * _IPT,), jnp.int32),
        [pltpu.VMEM((_LPG * _IPT, _OUT), jnp.float32) for _ in range(_NBUF)],
        pltpu.VMEM((_IPT, _OUT), jnp.float32),
        pltpu.VMEM((_OUT,), jnp.float32),
        [pltpu.SemaphoreType.DMA for _ in range(_NBUF)],
    ],
)
def _pool(xr_hbm, tp_hbm, b_hbm, out_hbm,
          idx_v, bufs, acc, bias_v, sems):
    _pool_body(xr_hbm, tp_hbm, b_hbm, out_hbm,
               idx_v, bufs, acc, bias_v, sems)


def kernel(x, table, W, b):
    x = x.astype(jnp.int32)
    tp = _project_table(table, W).reshape(_VOCAB, _OUT)
    xq = (x % _VS) * _PACK + x // _VS
    xr = xq.reshape(_NW, _IPT, _L).transpose(0, 2, 1).reshape(_NW, _L * _IPT)
    return _pool(xr, tp, b)


# speedup vs baseline: 2.5094x; 1.0027x over previous
import functools

import jax
import jax.numpy as jnp
from jax import lax
from jax.experimental import pallas as pl
from jax.experimental.pallas import tpu as pltpu
from jax.experimental.pallas import tpu_sc as plsc

_VOCAB = 100000
_DIM = 128
_OUT = 32
_B = 4096
_L = 200

_NC = 2
_NS = 16
_NW = _NC * _NS
_IPT = _B // _NW
_LANES = 16


def _bf16_bits(x):
    u = lax.bitcast_convert_type(x, jnp.int32)
    return (u + jnp.int32(0x7FFF) + ((u >> 16) & 1)) >> 16


_PACK = _DIM // _OUT
_VS = _VOCAB // _PACK


def _project_body(t0, t1, t2, t3, w_ref, o_ref):
    dn = (((1,), (1,)), ((), ()))
    for j, tr in enumerate((t0, t1, t2, t3)):
        o_ref[:, _OUT * j:_OUT * (j + 1)] = lax.dot_general(
            tr[...], w_ref[...], dimension_numbers=dn,
            preferred_element_type=jnp.float32)


def _project_table(table, W):
    rows_blk = 5000
    grid = _VS // rows_blk
    in_specs = [
        pl.BlockSpec((rows_blk, _DIM),
                     (lambda g, jj=j: (g + jj * grid, 0)))
        for j in range(_PACK)
    ]
    in_specs.append(pl.BlockSpec((_OUT, _DIM), lambda g: (0, 0)))
    return pl.pallas_call(
        _project_body,
        grid=(grid,),
        in_specs=in_specs,
        out_specs=pl.BlockSpec((rows_blk, _DIM), lambda g: (g, 0)),
        out_shape=jax.ShapeDtypeStruct((_VS, _DIM), jnp.float32),
    )(table, table, table, table, W)


_NBUF = 4
_LPG = 5


def _pool_body(xr_hbm, tp_hbm, b_hbm, out_hbm,
               idx_v, bufs, acc, bias_v, sems):
    wid = lax.axis_index("s") * _NC + lax.axis_index("c")
    pltpu.sync_copy(xr_hbm.at[wid], idx_v)
    pltpu.sync_copy(b_hbm, bias_v)

    zero = jnp.zeros((_LANES,), jnp.float32)

    def zr(r, c):
        acc[r, pl.ds(0, _LANES)] = zero
        acc[r, pl.ds(_LANES, _LANES)] = zero
        return c
    lax.fori_loop(0, _IPT, zr, 0, unroll=8)

    nsteps = _L // _LPG

    half = _LPG * _IPT // 2

    def _start(s, buf, sem):
        base = s * _LPG * _IPT
        pltpu.make_async_copy(
            tp_hbm.at[idx_v.at[pl.ds(base, half)]],
            buf.at[pl.ds(0, half), :], sem).start()
        pltpu.make_async_copy(
            tp_hbm.at[idx_v.at[pl.ds(base + half, half)]],
            buf.at[pl.ds(half, half), :], sem).start()

    def _wait(buf, sem):
        pltpu.make_async_copy(
            tp_hbm.at[idx_v.at[pl.ds(0, half)]],
            buf.at[pl.ds(0, half), :], sem).wait()
        pltpu.make_async_copy(
            tp_hbm.at[idx_v.at[pl.ds(0, half)]],
            buf.at[pl.ds(half, half), :], sem).wait()

    for j in range(_NBUF):
        _start(j, bufs[j], sems[j])

    def outer(i, c):
        s0 = i * _NBUF
        for j in range(_NBUF):
            s = s0 + j
            buf = bufs[j]
            sem = sems[j]
            _wait(buf, sem)

            def accum(r, cc):
                v0 = buf[r, pl.ds(0, _LANES)]
                v1 = buf[r, pl.ds(_LANES, _LANES)]
                for part in range(1, _LPG):
                    rr = part * _IPT + r
                    v0 = v0 + buf[rr, pl.ds(0, _LANES)]
                    v1 = v1 + buf[rr, pl.ds(_LANES, _LANES)]
                plsc.addupdate(acc.at[r, pl.ds(0, _LANES)], v0)
                plsc.addupdate(acc.at[r, pl.ds(_LANES, _LANES)], v1)
                return cc
            lax.fori_loop(0, _IPT, accum, 0, unroll=8)

            snext = s + _NBUF

            @pl.when(snext < nsteps)
            def _():
                _start(snext, buf, sem)
        return c
    lax.fori_loop(0, nsteps // _NBUF, outer, 0)

    scale = jnp.float32(1.0 / _L)
    blo = bias_v[pl.ds(0, _LANES)]
    bhi = bias_v[pl.ds(_LANES, _LANES)]

    def fin(r, c):
        v0 = acc[r, pl.ds(0, _LANES)] * scale + blo
        v1 = acc[r, pl.ds(_LANES, _LANES)] * scale + bhi
        acc[r, pl.ds(0, _LANES)] = 1.0 / (1.0 + jnp.exp(-v0))
        acc[r, pl.ds(_LANES, _LANES)] = 1.0 / (1.0 + jnp.exp(-v1))
        return c
    lax.fori_loop(0, _IPT, fin, 0, unroll=4)

    pltpu.sync_copy(acc, out_hbm.at[pl.ds(wid * _IPT, _IPT), :])


@functools.partial(
    pl.kernel,
    mesh=plsc.VectorSubcoreMesh(core_axis_name="c", subcore_axis_name="s"),
    compiler_params=pltpu.CompilerParams(
        use_tc_tiling_on_sc=False, needs_layout_passes=False),
    out_type=jax.ShapeDtypeStruct((_B, _OUT), jnp.float32),
    scratch_types=[
        pltpu.VMEM((_L * _IPT,), jnp.int32),
        [pltpu.VMEM((_LPG * _IPT, _OUT), jnp.float32) for _ in range(_NBUF)],
        pltpu.VMEM((_IPT, _OUT), jnp.float32),
        pltpu.VMEM((_OUT,), jnp.float32),
        [pltpu.SemaphoreType.DMA for _ in range(_NBUF)],
    ],
)
def _pool(xr_hbm, tp_hbm, b_hbm, out_hbm,
          idx_v, bufs, acc, bias_v, sems):
    _pool_body(xr_hbm, tp_hbm, b_hbm, out_hbm,
               idx_v, bufs, acc, bias_v, sems)


def kernel(x, table, W, b):
    x = x.astype(jnp.int32)
    tp = _project_table(table, W).reshape(_VOCAB, _OUT)
    xq = (x % _VS) * _PACK + x // _VS
    xr = xq.reshape(_NW, _IPT, _L).transpose(0, 2, 1).reshape(_NW, _L * _IPT)
    return _pool(xr, tp, b)
